# baseline (speedup 1.0000x reference)
import jax
import jax.numpy as jnp
from jax import lax
from jax.experimental import pallas as pl
from jax.experimental.pallas import tpu as pltpu
from jax.experimental.pallas import tpu_sc as plsc

CARD = 1000000
K = 16384
PAD_CARD = 1048576
NW = 32
CHUNK = PAD_CARD // NW
LOCAL_CAP = 1024
SORT_N = NW * LOCAL_CAP
INT_MIN = -2147483648


def _k1_body(freq_ref, gumbel_ref, key_ref, meta_ref, eqrow_ref):
    f = freq_ref[...]
    raw = 1.0 / (1.0 + f)
    mx = jnp.max(raw)
    e = jnp.exp(raw - mx)
    s = jnp.sum(e)
    probas = e / s
    logp = jnp.log(probas + 1e-20)
    scores = logp + gumbel_ref[...]
    b = lax.bitcast_convert_type(scores, jnp.int32)
    mkey = jnp.where(b >= 0, b, b ^ jnp.int32(0x7FFFFFFF))
    r = lax.broadcasted_iota(jnp.int32, mkey.shape, 0)
    c = lax.broadcasted_iota(jnp.int32, mkey.shape, 1)
    gidx = r * jnp.int32(mkey.shape[1]) + c
    mkey = jnp.where(gidx < CARD, mkey, jnp.int32(INT_MIN))
    key_ref[...] = mkey

    def body(t, c_acc):
        bit1 = lax.shift_left(jnp.int32(1), 31 - 2 * t)
        bit0 = lax.shift_left(jnp.int32(1), 30 - 2 * t)
        c10 = c_acc | bit1
        c11 = c10 | bit0
        c01 = c_acc | bit0
        flip = jnp.int32(INT_MIN)
        n10 = jnp.sum((mkey >= (c10 ^ flip)).astype(jnp.int32))
        n11 = jnp.sum((mkey >= (c11 ^ flip)).astype(jnp.int32))
        n01 = jnp.sum((mkey >= (c01 ^ flip)).astype(jnp.int32))
        hi = n10 >= K
        c_hi = jnp.where(hi, c10, c_acc)
        lo_cnt = jnp.where(hi, n11, n01)
        return jnp.where(lo_cnt >= K, c_hi | bit0, c_hi)

    c_final = lax.fori_loop(0, 16, body, jnp.int32(0))
    t_signed = c_final ^ jnp.int32(INT_MIN)
    m = jnp.sum((mkey > t_signed).astype(jnp.int32))
    quota = jnp.int32(K) - m
    mr = lax.broadcasted_iota(jnp.int32, meta_ref.shape, 0)
    meta_ref[...] = jnp.where(mr == 0, t_signed, quota)

    eqm = (mkey == t_signed).astype(jnp.int32)
    rows_per_chunk = CHUNK // mkey.shape[1]
    pr = lax.broadcasted_iota(jnp.int32, eqrow_ref.shape, 0)
    ptab = jnp.zeros(eqrow_ref.shape, jnp.int32)
    run = jnp.int32(0)
    for i in range(NW):
        ptab = jnp.where(pr == i, run, ptab)
        run = run + jnp.sum(eqm[i * rows_per_chunk:(i + 1) * rows_per_chunk])
    eqrow_ref[...] = ptab


def _run_k1(freq2d, gumbel2d):
    return pl.pallas_call(
        _k1_body,
        out_shape=(
            jax.ShapeDtypeStruct(freq2d.shape, jnp.int32),
            jax.ShapeDtypeStruct((8, 128), jnp.int32),
            jax.ShapeDtypeStruct((NW, 128), jnp.int32),
        ),
    )(freq2d, gumbel2d)


def _k2_body(keys_hbm, freq_hbm, tq_hbm, ptab_hbm,
             out_kv_hbm, out_iv_hbm, out_freq_hbm,
             mk_v, f_v, padk_v, padi_v, t_v, q_v, p_v, sem):
    wid = lax.axis_index("c") * 16 + lax.axis_index("s")
    base = wid * CHUNK
    lane = lax.broadcasted_iota(jnp.int32, (16,), 0)

    ck = pltpu.async_copy(keys_hbm.at[pl.ds(base, CHUNK)], mk_v, sem)
    cf = pltpu.async_copy(freq_hbm.at[pl.ds(base, CHUNK)], f_v, sem)

    pltpu.sync_copy(tq_hbm.at[0], t_v)
    pltpu.sync_copy(tq_hbm.at[1], q_v)
    pltpu.sync_copy(ptab_hbm.at[wid], p_v)
    t = t_v[pl.ds(0, 16)]
    quota = q_v[pl.ds(0, 16)]
    p_eq = p_v[pl.ds(0, 16)]

    sent_i = jnp.full((16,), PAD_CARD, jnp.int32) + wid * LOCAL_CAP + lane
    sent_k = jnp.full((16,), INT_MIN, jnp.int32)

    @plsc.parallel_loop(0, LOCAL_CAP // 16, 1, unroll=8, carry=sent_i)
    def _(j, si):
        padk_v[pl.ds(j * 16, 16)] = sent_k
        padi_v[pl.ds(j * 16, 16)] = si
        return si + 16

    ck.wait()
    cf.wait()

    one = jnp.ones((16,), jnp.float32)
    zero = jnp.zeros((16,), jnp.float32)
    base_vec = jnp.full((16,), base, jnp.int32)

    @plsc.parallel_loop(0, CHUNK // 16, 1, unroll=16,
                        carry=jnp.zeros((16,), jnp.int32))
    def _(j, sel_run):
        mk = mk_v[pl.ds(j * 16, 16)]
        sel = mk >= t
        pos = sel_run + plsc.cumsum(sel.astype(jnp.int32)) - 1
        pos = jnp.minimum(pos, LOCAL_CAP - 1)
        gidx = base_vec + j * 16 + lane
        plsc.store_scatter(padk_v, [pos], mk, mask=sel)
        plsc.store_scatter(padi_v, [pos], gidx, mask=sel)
        fv = f_v[pl.ds(j * 16, 16)]
        f_v[pl.ds(j * 16, 16)] = fv + jnp.where(mk > t, one, zero)
        return sel_run + plsc.all_reduce_population_count(sel)

    def tbody(j, eq_run):
        pk = padk_v[pl.ds(j * 16, 16)]
        em = pk == t
        tie_rank = p_eq + eq_run + plsc.cumsum(em.astype(jnp.int32)) - 1
        tsel = em & (tie_rank < quota)
        off = padi_v[pl.ds(j * 16, 16)] - base_vec
        off = jnp.clip(off, 0, CHUNK - 1)
        fg = plsc.load_gather(f_v, [off], mask=tsel)
        plsc.store_scatter(f_v, [off], fg + one, mask=tsel)
        return eq_run + plsc.all_reduce_population_count(em)

    _ = lax.fori_loop(0, LOCAL_CAP // 16, tbody, jnp.zeros((16,), jnp.int32))

    n_full = CARD // CHUNK
    rem = CARD - n_full * CHUNK

    @pl.when(wid < n_full)
    def _():
        pltpu.sync_copy(f_v, out_freq_hbm.at[pl.ds(base, CHUNK)])

    @pl.when(wid == n_full)
    def _():
        pltpu.sync_copy(f_v.at[pl.ds(0, rem)],
                        out_freq_hbm.at[pl.ds(n_full * CHUNK, rem)])

    pltpu.sync_copy(padk_v, out_kv_hbm.at[pl.ds(wid * LOCAL_CAP, LOCAL_CAP)])
    pltpu.sync_copy(padi_v, out_iv_hbm.at[pl.ds(wid * LOCAL_CAP, LOCAL_CAP)])


def _run_k2(mkeys_flat, freq_pad, tq, ptab):
    mesh = plsc.VectorSubcoreMesh(
        core_axis_name="c", subcore_axis_name="s", num_cores=2)
    fn = pl.kernel(
        _k2_body,
        compiler_params=pltpu.CompilerParams(needs_layout_passes=False),
        out_type=(
            jax.ShapeDtypeStruct((SORT_N,), jnp.int32),
            jax.ShapeDtypeStruct((SORT_N,), jnp.int32),
            jax.ShapeDtypeStruct((CARD,), jnp.float32),
        ),
        mesh=mesh,
        scratch_types=[
            pltpu.VMEM((CHUNK,), jnp.int32),
            pltpu.VMEM((CHUNK,), jnp.float32),
            pltpu.VMEM((LOCAL_CAP,), jnp.int32),
            pltpu.VMEM((LOCAL_CAP,), jnp.int32),
            pltpu.VMEM((128,), jnp.int32),
            pltpu.VMEM((128,), jnp.int32),
            pltpu.VMEM((128,), jnp.int32),
            pltpu.SemaphoreType.DMA,
        ],
    )
    return fn(mkeys_flat, freq_pad, tq, ptab)


def _k3_body(key_ref, idx_ref, out_ref):
    rows = SORT_N // 128
    xk = key_ref[...]
    xi = idx_ref[...]
    rr = lax.broadcasted_iota(jnp.int32, (rows, 128), 0)
    cc = lax.broadcasted_iota(jnp.int32, (rows, 128), 1)
    jj = rr * 128 + cc

    def cmp_exchange(xk, xi, d, p):
        if d < 128:
            ax, s, n = 1, d, 128
        else:
            ax, s, n = 0, d // 128, rows
        pk_m = pltpu.roll(xk, n - s, ax)
        pk_p = pltpu.roll(xk, s, ax)
        pi_m = pltpu.roll(xi, n - s, ax)
        pi_p = pltpu.roll(xi, s, ax)
        upper = (jj & d) != 0
        pk = jnp.where(upper, pk_p, pk_m)
        pi = jnp.where(upper, pi_p, pi_m)
        dirbit = (jj & (1 << (p + 1))) == 0
        want_small = jnp.logical_xor(upper, dirbit)
        less = (xk > pk) | ((xk == pk) & (xi < pi))
        keep = less == want_small
        return jnp.where(keep, xk, pk), jnp.where(keep, xi, pi)

    log_n = SORT_N.bit_length() - 1
    for p in range(log_n):
        for q in range(p, -1, -1):
            xk, xi = cmp_exchange(xk, xi, 1 << q, p)
    out_ref[...] = xi


def _run_k3(keys, idxs):
    rows = SORT_N // 128
    return pl.pallas_call(
        _k3_body,
        out_shape=jax.ShapeDtypeStruct((rows, 128), jnp.int32),
    )(keys.reshape(rows, 128), idxs.reshape(rows, 128))


@jax.jit
def kernel(item_id, frequencies):
    freq_pad = jnp.pad(frequencies, (0, PAD_CARD - CARD))
    freq2d = freq_pad.reshape(1024, 1024)

    skey = jax.random.key(42)
    u = jax.random.uniform(skey, (CARD,), minval=1e-9, maxval=1.0)
    gumbel = -jnp.log(-jnp.log(u))
    gumbel2d = jnp.pad(gumbel, (0, PAD_CARD - CARD)).reshape(1024, 1024)

    mkeys, tq, ptab = _run_k1(freq2d, gumbel2d)

    pad_keys, pad_idx, new_freq = _run_k2(
        mkeys.reshape(PAD_CARD), freq_pad, tq, ptab)

    sorted_idx = _run_k3(pad_keys, pad_idx)
    negatives = sorted_idx.reshape(SORT_N)[:K]
    return (item_id, negatives, new_freq)


# speedup vs baseline: 1.0536x; 1.0536x over previous
import jax
import jax.numpy as jnp
from jax import lax
from jax.experimental import pallas as pl
from jax.experimental.pallas import tpu as pltpu
from jax.experimental.pallas import tpu_sc as plsc

CARD = 1000000
K = 16384
PAD_CARD = 1048576
NW = 32
CHUNK = PAD_CARD // NW
LOCAL_CAP = 1024
SORT_N = NW * LOCAL_CAP
INT_MIN = -2147483648


def _k1_body(freq_ref, gumbel_ref, key_ref, meta_ref, eqrow_ref):
    f = freq_ref[...]
    raw = 1.0 / (1.0 + f)
    mx = jnp.max(raw)
    e = jnp.exp(raw - mx)
    s = jnp.sum(e)
    probas = e / s
    logp = jnp.log(probas + 1e-20)
    scores = logp + gumbel_ref[...]
    b = lax.bitcast_convert_type(scores, jnp.int32)
    mkey = jnp.where(b >= 0, b, b ^ jnp.int32(0x7FFFFFFF))
    r = lax.broadcasted_iota(jnp.int32, mkey.shape, 0)
    c = lax.broadcasted_iota(jnp.int32, mkey.shape, 1)
    gidx = r * jnp.int32(mkey.shape[1]) + c
    mkey = jnp.where(gidx < CARD, mkey, jnp.int32(INT_MIN))
    key_ref[...] = mkey

    def body(t, c_acc):
        bit1 = lax.shift_left(jnp.int32(1), 31 - 2 * t)
        bit0 = lax.shift_left(jnp.int32(1), 30 - 2 * t)
        c10 = c_acc | bit1
        c11 = c10 | bit0
        c01 = c_acc | bit0
        flip = jnp.int32(INT_MIN)
        n10 = jnp.sum((mkey >= (c10 ^ flip)).astype(jnp.int32))
        n11 = jnp.sum((mkey >= (c11 ^ flip)).astype(jnp.int32))
        n01 = jnp.sum((mkey >= (c01 ^ flip)).astype(jnp.int32))
        hi = n10 >= K
        c_hi = jnp.where(hi, c10, c_acc)
        lo_cnt = jnp.where(hi, n11, n01)
        return jnp.where(lo_cnt >= K, c_hi | bit0, c_hi)

    c_final = lax.fori_loop(0, 16, body, jnp.int32(0))
    t_signed = c_final ^ jnp.int32(INT_MIN)
    m = jnp.sum((mkey > t_signed).astype(jnp.int32))
    quota = jnp.int32(K) - m
    mr = lax.broadcasted_iota(jnp.int32, meta_ref.shape, 0)
    meta_ref[...] = jnp.where(mr == 0, t_signed, quota)

    eqm = (mkey == t_signed).astype(jnp.int32)
    rows_per_chunk = CHUNK // mkey.shape[1]
    pr = lax.broadcasted_iota(jnp.int32, eqrow_ref.shape, 0)
    ptab = jnp.zeros(eqrow_ref.shape, jnp.int32)
    run = jnp.int32(0)
    for i in range(NW):
        ptab = jnp.where(pr == i, run, ptab)
        run = run + jnp.sum(eqm[i * rows_per_chunk:(i + 1) * rows_per_chunk])
    eqrow_ref[...] = ptab


def _run_k1(freq2d, gumbel2d):
    return pl.pallas_call(
        _k1_body,
        out_shape=(
            jax.ShapeDtypeStruct(freq2d.shape, jnp.int32),
            jax.ShapeDtypeStruct((8, 128), jnp.int32),
            jax.ShapeDtypeStruct((NW, 128), jnp.int32),
        ),
    )(freq2d, gumbel2d)


def _k2_body(keys_hbm, freq_hbm, tq_hbm, ptab_hbm,
             out_kv_hbm, out_iv_hbm, out_freq_hbm,
             mk_v, f_v, padk_v, padi_v, t_v, q_v, p_v, sem):
    wid = lax.axis_index("c") * 16 + lax.axis_index("s")
    base = wid * CHUNK
    lane = lax.broadcasted_iota(jnp.int32, (16,), 0)

    ck = pltpu.async_copy(keys_hbm.at[pl.ds(base, CHUNK)], mk_v, sem)
    cf = pltpu.async_copy(freq_hbm.at[pl.ds(base, CHUNK)], f_v, sem)

    pltpu.sync_copy(tq_hbm.at[0], t_v)
    pltpu.sync_copy(tq_hbm.at[1], q_v)
    pltpu.sync_copy(ptab_hbm.at[wid], p_v)
    t = t_v[pl.ds(0, 16)]
    quota = q_v[pl.ds(0, 16)]
    p_eq = p_v[pl.ds(0, 16)]

    sent_i = jnp.full((16,), PAD_CARD, jnp.int32) + wid * LOCAL_CAP + lane
    sent_k = jnp.full((16,), INT_MIN, jnp.int32)

    @plsc.parallel_loop(0, LOCAL_CAP // 16, 1, unroll=8, carry=sent_i)
    def _(j, si):
        padk_v[pl.ds(j * 16, 16)] = sent_k
        padi_v[pl.ds(j * 16, 16)] = si
        return si + 16

    ck.wait()
    cf.wait()

    one = jnp.ones((16,), jnp.float32)
    zero = jnp.zeros((16,), jnp.float32)
    base_vec = jnp.full((16,), base, jnp.int32)

    @plsc.parallel_loop(0, CHUNK // 16, 1, unroll=8,
                        carry=jnp.zeros((16,), jnp.int32))
    def _(j, sel_run):
        mk = mk_v[pl.ds(j * 16, 16)]
        sel = mk >= t
        pos = sel_run + plsc.cumsum(sel.astype(jnp.int32)) - 1
        pos = jnp.minimum(pos, LOCAL_CAP - 1)
        gidx = base_vec + j * 16 + lane
        plsc.store_scatter(padk_v, [pos], mk, mask=sel)
        plsc.store_scatter(padi_v, [pos], gidx, mask=sel)
        fv = f_v[pl.ds(j * 16, 16)]
        f_v[pl.ds(j * 16, 16)] = fv + jnp.where(mk > t, one, zero)
        return sel_run + plsc.all_reduce_population_count(sel)

    def tbody(j, eq_run):
        pk = padk_v[pl.ds(j * 16, 16)]
        em = pk == t
        tie_rank = p_eq + eq_run + plsc.cumsum(em.astype(jnp.int32)) - 1
        tsel = em & (tie_rank < quota)
        off = padi_v[pl.ds(j * 16, 16)] - base_vec
        off = jnp.clip(off, 0, CHUNK - 1)
        fg = plsc.load_gather(f_v, [off], mask=tsel)
        plsc.store_scatter(f_v, [off], fg + one, mask=tsel)
        return eq_run + plsc.all_reduce_population_count(em)

    _ = lax.fori_loop(0, LOCAL_CAP // 16, tbody, jnp.zeros((16,), jnp.int32))

    n_full = CARD // CHUNK
    rem = CARD - n_full * CHUNK

    @pl.when(wid < n_full)
    def _():
        pltpu.sync_copy(f_v, out_freq_hbm.at[pl.ds(base, CHUNK)])

    @pl.when(wid == n_full)
    def _():
        pltpu.sync_copy(f_v.at[pl.ds(0, rem)],
                        out_freq_hbm.at[pl.ds(n_full * CHUNK, rem)])

    pltpu.sync_copy(padk_v, out_kv_hbm.at[pl.ds(wid * LOCAL_CAP, LOCAL_CAP)])
    pltpu.sync_copy(padi_v, out_iv_hbm.at[pl.ds(wid * LOCAL_CAP, LOCAL_CAP)])


def _run_k2(mkeys_flat, freq_pad, tq, ptab):
    mesh = plsc.VectorSubcoreMesh(
        core_axis_name="c", subcore_axis_name="s", num_cores=2)
    fn = pl.kernel(
        _k2_body,
        compiler_params=pltpu.CompilerParams(needs_layout_passes=False),
        out_type=(
            jax.ShapeDtypeStruct((SORT_N,), jnp.int32),
            jax.ShapeDtypeStruct((SORT_N,), jnp.int32),
            jax.ShapeDtypeStruct((CARD,), jnp.float32),
        ),
        mesh=mesh,
        scratch_types=[
            pltpu.VMEM((CHUNK,), jnp.int32),
            pltpu.VMEM((CHUNK,), jnp.float32),
            pltpu.VMEM((LOCAL_CAP,), jnp.int32),
            pltpu.VMEM((LOCAL_CAP,), jnp.int32),
            pltpu.VMEM((128,), jnp.int32),
            pltpu.VMEM((128,), jnp.int32),
            pltpu.VMEM((128,), jnp.int32),
            pltpu.SemaphoreType.DMA,
        ],
    )
    return fn(mkeys_flat, freq_pad, tq, ptab)


def _k3_body(key_ref, idx_ref, out_ref):
    rows = SORT_N // 128
    xk = key_ref[...]
    xi = idx_ref[...]
    rr = lax.broadcasted_iota(jnp.int32, (rows, 128), 0)
    cc = lax.broadcasted_iota(jnp.int32, (rows, 128), 1)
    jj = rr * 128 + cc

    def cmp_exchange(xk, xi, d, p):
        if d < 128:
            ax, s, n = 1, d, 128
        else:
            ax, s, n = 0, d // 128, rows
        pk_m = pltpu.roll(xk, n - s, ax)
        pk_p = pltpu.roll(xk, s, ax)
        pi_m = pltpu.roll(xi, n - s, ax)
        pi_p = pltpu.roll(xi, s, ax)
        upper = (jj & d) != 0
        pk = jnp.where(upper, pk_p, pk_m)
        pi = jnp.where(upper, pi_p, pi_m)
        dirbit = (jj & (1 << (p + 1))) == 0
        want_small = jnp.logical_xor(upper, dirbit)
        less = (xk > pk) | ((xk == pk) & (xi < pi))
        keep = less == want_small
        return jnp.where(keep, xk, pk), jnp.where(keep, xi, pi)

    log_n = SORT_N.bit_length() - 1
    for p in range(log_n):
        for q in range(p, -1, -1):
            xk, xi = cmp_exchange(xk, xi, 1 << q, p)
    out_ref[...] = xi


def _run_k3(keys, idxs):
    rows = SORT_N // 128
    return pl.pallas_call(
        _k3_body,
        out_shape=jax.ShapeDtypeStruct((rows, 128), jnp.int32),
    )(keys.reshape(rows, 128), idxs.reshape(rows, 128))


@jax.jit
def kernel(item_id, frequencies):
    freq_pad = jnp.pad(frequencies, (0, PAD_CARD - CARD))
    freq2d = freq_pad.reshape(1024, 1024)

    skey = jax.random.key(42)
    u = jax.random.uniform(skey, (CARD,), minval=1e-9, maxval=1.0)
    gumbel = -jnp.log(-jnp.log(u))
    gumbel2d = jnp.pad(gumbel, (0, PAD_CARD - CARD)).reshape(1024, 1024)

    mkeys, tq, ptab = _run_k1(freq2d, gumbel2d)

    pad_keys, pad_idx, new_freq = _run_k2(
        mkeys.reshape(PAD_CARD), freq_pad, tq, ptab)

    sorted_idx = _run_k3(pad_keys, pad_idx)
    negatives = sorted_idx.reshape(SORT_N)[:K]
    return (item_id, negatives, new_freq)
